# Initial kernel scaffold; baseline (speedup 1.0000x reference)
import jax
import jax.numpy as jnp
from jax.experimental import pallas as pl


def kernel(input, idx1, idx2):
    raise NotImplementedError("write your pallas kernel here")


# speedup vs baseline: 3.1475x; 3.1475x over previous
import jax
import jax.numpy as jnp
from jax import lax
from jax.experimental import pallas as pl
from jax.experimental.pallas import tpu as pltpu
from jax.experimental.pallas import tpu_sc as plsc

L, B, S, D = 13, 4, 2048, 768
NROWS = 3 * B
NLANES = 16
LAST_BASE = (L - 1) * B * S

SRC_VEC = tuple((l // 3 if l % 3 == 1 else 4 + l // 3 if l % 3 == 2 else 8)
                if l < NROWS else 8 for l in range(NLANES))
BASE_VEC = tuple(LAST_BASE + (l // 3) * S if l < NROWS else 0
                 for l in range(NLANES))


def _sc_gather(table, idx_all, src_pos, base_row):
    mesh = plsc.VectorSubcoreMesh(core_axis_name="c", subcore_axis_name="s")

    @pl.kernel(
        mesh=mesh,
        out_type=jax.ShapeDtypeStruct((NLANES, D), jnp.float32),
        scratch_types=[
            pltpu.VMEM((NLANES,), jnp.int32),
            pltpu.VMEM((NLANES,), jnp.int32),
            pltpu.VMEM((NLANES,), jnp.int32),
            pltpu.VMEM((NLANES, D), jnp.float32),
            pltpu.SemaphoreType.DMA,
        ],
    )
    def k(table_hbm, idx_hbm, src_hbm, base_hbm, out_hbm,
          idx_v, src_v, base_v, rows_v, sem):
        is_w0 = (lax.axis_index("c") == 0) & (lax.axis_index("s") == 0)

        @pl.when(is_w0)
        def _():
            pltpu.sync_copy(idx_hbm, idx_v)
            pltpu.sync_copy(src_hbm, src_v)
            pltpu.sync_copy(base_hbm, base_v)
            tokens = lax.gather(
                idx_v[...], src_v[...][:, None],
                lax.GatherDimensionNumbers(
                    offset_dims=(), collapsed_slice_dims=(0,),
                    start_index_map=(0,)),
                slice_sizes=(1,),
                mode=lax.GatherScatterMode.PROMISE_IN_BOUNDS)
            base_v[...] = base_v[...] + tokens
            pltpu.async_copy(table_hbm.at[base_v], rows_v, sem).wait()
            pltpu.sync_copy(rows_v, out_hbm)

    return k(table, idx_all, src_pos, base_row)


def kernel(input, idx1, idx2):
    table = input.reshape(L * B * S, D)
    idx_all = jnp.concatenate(
        [idx1, idx2, jnp.zeros((NLANES - 2 * B,), jnp.int32)])
    src_pos = jnp.asarray(SRC_VEC, jnp.int32)
    base_row = jnp.asarray(BASE_VEC, jnp.int32)
    out = _sc_gather(table, idx_all, src_pos, base_row)
    return out[:NROWS].reshape(B, 3 * D)


# speedup vs baseline: 3.6658x; 1.1647x over previous
import jax
import jax.numpy as jnp
from jax import lax
from jax.experimental import pallas as pl
from jax.experimental.pallas import tpu as pltpu
from jax.experimental.pallas import tpu_sc as plsc

L, B, S, D = 13, 4, 2048, 768
NROWS = 3 * B
NLANES = 16
LAST_BASE = (L - 1) * B * S

SRC_VEC = tuple((l // 3 if l % 3 == 1 else 4 + l // 3 if l % 3 == 2 else 8)
                if l < NROWS else 8 for l in range(NLANES))
BASE_VEC = tuple(LAST_BASE + (l // 3) * S if l < NROWS else 0
                 for l in range(NLANES))


def _sc_gather(table, packed):
    mesh = plsc.VectorSubcoreMesh(
        core_axis_name="c", subcore_axis_name="s", num_cores=1)

    @pl.kernel(
        mesh=mesh,
        out_type=jax.ShapeDtypeStruct((NLANES, D), jnp.float32),
        scratch_types=[
            pltpu.VMEM((3 * NLANES,), jnp.int32),
            pltpu.VMEM((NLANES, D), jnp.float32),
            pltpu.SemaphoreType.DMA,
        ],
    )
    def k(table_hbm, packed_hbm, out_hbm, packed_v, rows_v, sem):
        is_w0 = lax.axis_index("s") == 0

        @pl.when(is_w0)
        def _():
            pltpu.sync_copy(packed_hbm, packed_v)
            idx = packed_v[pl.ds(0, NLANES)]
            src = packed_v[pl.ds(NLANES, NLANES)]
            base = packed_v[pl.ds(2 * NLANES, NLANES)]
            tokens = lax.gather(
                idx, src[:, None],
                lax.GatherDimensionNumbers(
                    offset_dims=(), collapsed_slice_dims=(0,),
                    start_index_map=(0,)),
                slice_sizes=(1,),
                mode=lax.GatherScatterMode.PROMISE_IN_BOUNDS)
            pltpu.async_copy(table_hbm.at[base + tokens], rows_v, sem).wait()
            pltpu.sync_copy(rows_v, out_hbm)

    return k(table, packed)


def kernel(input, idx1, idx2):
    table = input.reshape(L * B * S, D)
    packed = jnp.concatenate([
        idx1, idx2, jnp.zeros((NLANES - 2 * B,), jnp.int32),
        jnp.asarray(SRC_VEC, jnp.int32), jnp.asarray(BASE_VEC, jnp.int32)])
    out = _sc_gather(table, packed)
    return out[:NROWS].reshape(B, 3 * D)


# speedup vs baseline: 3.7184x; 1.0143x over previous
import jax
import jax.numpy as jnp
from jax import lax
from jax.experimental import pallas as pl
from jax.experimental.pallas import tpu as pltpu
from jax.experimental.pallas import tpu_sc as plsc

L, B, S, D = 13, 4, 2048, 768
NROWS = 3 * B
NLANES = 16
LAST_BASE = (L - 1) * B * S

ROW_OF = tuple(min(l, NROWS - 1) for l in range(NLANES))
SRC_VEC = tuple((r // 3 if r % 3 == 1 else 8 + r // 3 if r % 3 == 2 else 0)
                for r in ROW_OF)
MASK_VEC = tuple(0 if r % 3 == 0 else 1 for r in ROW_OF)
BASE_VEC = tuple(LAST_BASE + (r // 3) * S for r in ROW_OF)
CONST_BLOCK = SRC_VEC + BASE_VEC + MASK_VEC


def _sc_gather(table, idx1, idx2, consts):
    mesh = plsc.VectorSubcoreMesh(
        core_axis_name="c", subcore_axis_name="s", num_cores=1)

    @pl.kernel(
        mesh=mesh,
        out_type=jax.ShapeDtypeStruct((NROWS, D), jnp.float32),
        scratch_types=[
            pltpu.VMEM((3 * NLANES,), jnp.int32),
            pltpu.VMEM((NLANES,), jnp.int32),
            pltpu.VMEM((NLANES,), jnp.int32),
            pltpu.VMEM((8, D), jnp.float32),
            pltpu.VMEM((B, D), jnp.float32),
            pltpu.SemaphoreType.DMA,
        ],
    )
    def k(table_hbm, idx1_hbm, idx2_hbm, consts_hbm, out_hbm,
          const_v, idx_v, ridx_v, rows_lo, rows_hi, sem):
        is_w0 = lax.axis_index("s") == 0

        @pl.when(is_w0)
        def _():
            cp_c = pltpu.async_copy(consts_hbm, const_v, sem)
            cp_1 = pltpu.async_copy(idx1_hbm, idx_v.at[pl.ds(0, B)], sem)
            cp_2 = pltpu.async_copy(idx2_hbm, idx_v.at[pl.ds(8, B)], sem)
            cp_c.wait()
            cp_1.wait()
            cp_2.wait()
            src = const_v[pl.ds(0, NLANES)]
            base = const_v[pl.ds(NLANES, NLANES)]
            msk = const_v[pl.ds(2 * NLANES, NLANES)]
            tokens = lax.gather(
                idx_v[...], src[:, None],
                lax.GatherDimensionNumbers(
                    offset_dims=(), collapsed_slice_dims=(0,),
                    start_index_map=(0,)),
                slice_sizes=(1,),
                mode=lax.GatherScatterMode.PROMISE_IN_BOUNDS)
            ridx_v[...] = base + tokens * msk
            g_lo = pltpu.async_copy(
                table_hbm.at[ridx_v.at[pl.ds(0, 8)]], rows_lo, sem)
            g_hi = pltpu.async_copy(
                table_hbm.at[ridx_v.at[pl.ds(8, B)]], rows_hi, sem)
            g_lo.wait()
            g_hi.wait()
            o_lo = pltpu.async_copy(
                rows_lo, out_hbm.at[pl.ds(0, 8)], sem)
            o_hi = pltpu.async_copy(
                rows_hi, out_hbm.at[pl.ds(8, B)], sem)
            o_lo.wait()
            o_hi.wait()

    return k(table, idx1, idx2, consts)


def kernel(input, idx1, idx2):
    table = input.reshape(L * B * S, D)
    consts = jnp.asarray(CONST_BLOCK, jnp.int32)
    out = _sc_gather(table, idx1, idx2, consts)
    return out.reshape(B, 3 * D)
